# Initial kernel scaffold; baseline (speedup 1.0000x reference)
import jax
import jax.numpy as jnp
from jax.experimental import pallas as pl


def kernel(x0, x1, W1, b1, W2, b2, a1_w, a1_b, a2_w, a2_b, edge_index):
    raise NotImplementedError("write your pallas kernel here")


# speedup vs baseline: 13.0734x; 13.0734x over previous
import functools

import jax
import jax.numpy as jnp
from jax import lax
from jax.experimental import pallas as pl
from jax.experimental.pallas import tpu as pltpu
from jax.experimental.pallas import tpu_sc as plsc

_N = 10000
_E = 320000
_D = 128

_NC = 2
_NS = 16
_NW = _NC * _NS
_EPT = _E // _NW
_B = 80
_NB = _EPT // _B
_NP = 10240
_RPT = _NP // _NS


def _dense_body(x_ref, w1_ref, b1_ref, w2_ref, b2_ref, a1w_ref, a1b_ref,
                a2w_ref, a2b_ref, xi_ref, xj_ref, ai_ref, aj_ref):
    x = x_ref[:]
    dn = (((1,), (1,)), ((), ()))
    xi = jnp.maximum(
        lax.dot_general(x, w1_ref[:], dn, preferred_element_type=jnp.float32)
        + b1_ref[:], 0.0)
    xj = jnp.maximum(
        lax.dot_general(x, w2_ref[:], dn, preferred_element_type=jnp.float32)
        + b2_ref[:], 0.0)
    xi_ref[:] = xi
    xj_ref[:] = xj
    ai_ref[:] = jnp.dot(xi, a1w_ref[:],
                        preferred_element_type=jnp.float32) + a1b_ref[:]
    aj_ref[:] = jnp.dot(xj, a2w_ref[:],
                        preferred_element_type=jnp.float32) + a2b_ref[:]


def _dense(x0, W1, b1, W2, b2, a1_w, a1_b, a2_w, a2_b):
    bn = 1000
    grid = (_N // bn,)
    full = lambda shape: pl.BlockSpec(shape, lambda i: (0, 0))
    rows = lambda shape: pl.BlockSpec(shape, lambda i: (i, 0))
    return pl.pallas_call(
        _dense_body,
        grid=grid,
        in_specs=[
            rows((bn, _D)),
            full((_D, _D)), full((1, _D)),
            full((_D, _D)), full((1, _D)),
            full((_D, 1)), full((1, 1)),
            full((_D, 1)), full((1, 1)),
        ],
        out_specs=[
            rows((bn, _D)), rows((bn, _D)),
            rows((bn, 1)), rows((bn, 1)),
        ],
        out_shape=[
            jax.ShapeDtypeStruct((_N, _D), jnp.float32),
            jax.ShapeDtypeStruct((_N, _D), jnp.float32),
            jax.ShapeDtypeStruct((_N, 1), jnp.float32),
            jax.ShapeDtypeStruct((_N, 1), jnp.float32),
        ],
    )(x0, W1, b1.reshape(1, _D), W2, b2.reshape(1, _D),
      a1_w, a1_b.reshape(1, 1), a2_w, a2_b.reshape(1, 1))


def _edge_body(xj_hbm, ai_hbm, aj_hbm, row_hbm, col_hbm, zer_hbm,
               out0, out1,
               acc, ai_v, aj_v, ridx, cidx, att, rows, sem):
    c = lax.axis_index("c")
    s = lax.axis_index("s")
    w = s * _NC + c

    pltpu.sync_copy(ai_hbm, ai_v)
    pltpu.sync_copy(aj_hbm, aj_v)

    pltpu.sync_copy(zer_hbm, acc.at[pl.ds(s * _RPT, _RPT)])
    plsc.subcore_barrier()

    def block(blk, carry):
        base = w * _EPT + blk * _B
        pltpu.sync_copy(row_hbm.at[pl.ds(base, _B)], ridx)
        pltpu.sync_copy(col_hbm.at[pl.ds(base, _B)], cidx)
        pltpu.async_copy(xj_hbm.at[cidx], rows, sem).wait()
        for v in range(_B // 16):
            sl = pl.ds(v * 16, 16)
            r = ridx[sl]
            cc = cidx[sl]
            a = plsc.load_gather(ai_v, [r])
            b = plsc.load_gather(aj_v, [cc])
            att[sl] = 1.0 / (1.0 + jnp.exp(-(a + b)))

        def escale(e, cy):
            sv = plsc.load_gather(att, [jnp.full((16,), e, jnp.int32)])
            for j in range(_D // 16):
                fsl = pl.ds(j * 16, 16)
                rows[e, fsl] = rows[e, fsl] * sv
            return cy

        lax.fori_loop(0, _B, escale, 0)
        pltpu.sync_copy(rows, acc.at[ridx], add=True)
        return carry

    lax.fori_loop(0, _NB, block, 0)
    plsc.subcore_barrier()

    osl = pl.ds(s * _RPT, _RPT)

    @pl.when(c == 0)
    def _():
        pltpu.sync_copy(acc.at[osl], out0.at[osl])

    @pl.when(c == 1)
    def _():
        pltpu.sync_copy(acc.at[osl], out1.at[osl])


@functools.partial(
    pl.kernel,
    out_type=[jax.ShapeDtypeStruct((_NP, _D), jnp.float32)] * 2,
    mesh=plsc.VectorSubcoreMesh(core_axis_name="c", subcore_axis_name="s"),
    compiler_params=pltpu.CompilerParams(needs_layout_passes=False),
    scratch_types=[
        pltpu.VMEM_SHARED((_NP, _D), jnp.float32),
        pltpu.VMEM((_N,), jnp.float32),
        pltpu.VMEM((_N,), jnp.float32),
        pltpu.VMEM((_B,), jnp.int32),
        pltpu.VMEM((_B,), jnp.int32),
        pltpu.VMEM((_B,), jnp.float32),
        pltpu.VMEM((_B, _D), jnp.float32),
        pltpu.SemaphoreType.DMA,
    ],
)
def _edges(xj_hbm, ai_hbm, aj_hbm, row_hbm, col_hbm, zer_hbm, out0, out1,
           acc, ai_v, aj_v, ridx, cidx, att, rows, sem):
    _edge_body(xj_hbm, ai_hbm, aj_hbm, row_hbm, col_hbm, zer_hbm,
               out0, out1, acc, ai_v, aj_v, ridx, cidx, att, rows, sem)


def _add_body(xi_ref, p0_ref, p1_ref, o_ref):
    o_ref[:] = xi_ref[:] + p0_ref[:] + p1_ref[:]


def _final_add(xi, p0, p1):
    bn = 1000
    spec = pl.BlockSpec((bn, _D), lambda i: (i, 0))
    return pl.pallas_call(
        _add_body,
        grid=(_N // bn,),
        in_specs=[spec, spec, spec],
        out_specs=spec,
        out_shape=jax.ShapeDtypeStruct((_N, _D), jnp.float32),
    )(xi, p0, p1)


def kernel(x0, x1, W1, b1, W2, b2, a1_w, a1_b, a2_w, a2_b, edge_index):
    xi, xj, ai, aj = _dense(x0, W1, b1, W2, b2, a1_w, a1_b, a2_w, a2_b)
    ai = ai.reshape(_N)
    aj = aj.reshape(_N)
    row = edge_index[0]
    col = edge_index[1]
    zer = jnp.zeros((_RPT, _D), jnp.float32)
    p0, p1 = _edges(xj, ai, aj, row, col, zer)
    return _final_add(xi, p0, p1)
